# baseline (speedup 1.0000x reference)
import jax
import jax.numpy as jnp
from jax.experimental import pallas as pl
from jax.experimental.pallas import tpu as pltpu

DIM = 1024
DEPTH = 3
NUM_EXPERTS = 2 ** DEPTH
NUM_ROUTERS = NUM_EXPERTS - 1
HIDDEN = DIM * 4
HBLK = 1024


def _ffff_kernel(x_ref, rw_ref, rb_ref, w1_ref, b1_ref, w2_ref, b2_ref,
                 out_ref, p_ref, pe_ref):
    e = pl.program_id(0)
    k = pl.program_id(1)

    @pl.when((e == 0) & (k == 0))
    def _init():
        x = x_ref[...]
        logits = jax.lax.dot_general(
            x, rw_ref[...], (((1,), (1,)), ((), ())),
            preferred_element_type=jnp.float32) + rb_ref[...]
        s = jax.nn.sigmoid(logits)
        cols = []
        for ei in range(NUM_EXPERTS):
            bit0 = (ei >> 2) & 1
            bit1 = (ei >> 1) & 1
            bit2 = ei & 1
            c1 = 1 + (ei >> 2)
            c2 = 3 + (ei >> 1)
            t0 = s[:, 0:1] if bit0 else 1.0 - s[:, 0:1]
            t1 = s[:, c1:c1 + 1] if bit1 else 1.0 - s[:, c1:c1 + 1]
            t2 = s[:, c2:c2 + 1] if bit2 else 1.0 - s[:, c2:c2 + 1]
            cols.append(t0 * t1 * t2)
        p_ref[...] = jnp.concatenate(cols, axis=1)
        out_ref[...] = jnp.zeros_like(out_ref)

    @pl.when(k == 0)
    def _per_expert():
        lanes = jax.lax.broadcasted_iota(jnp.int32, (1, NUM_EXPERTS), 1)
        pe = jnp.sum(p_ref[...] * (lanes == e).astype(jnp.float32),
                     axis=1, keepdims=True)
        pe_ref[...] = pe
        out_ref[...] += pe * b2_ref[0]

    x = x_ref[...]
    pe = pe_ref[...]
    nsub = 4
    sub = HBLK // nsub
    partials = []
    for i in range(nsub):
        sl = slice(i * sub, (i + 1) * sub)
        h = jnp.dot(x, w1_ref[0, :, sl], preferred_element_type=jnp.float32)
        h = h + b1_ref[0, :, sl]
        h = 0.5 * h * (1.0 + jax.lax.erf(h * 0.7071067811865476))
        h = h * pe
        partials.append(jnp.dot(h, w2_ref[0, sl, :],
                                preferred_element_type=jnp.float32))
    out_ref[...] += ((partials[0] + partials[1]) +
                     (partials[2] + partials[3]))


def kernel(x, router_w, router_b, w1, b1, w2, b2):
    batch, seq, dim = x.shape
    n = batch * seq
    flat_x = x.reshape(n, dim)
    rb2d = router_b.reshape(1, NUM_ROUTERS)
    b1r = b1.reshape(NUM_EXPERTS, 1, HIDDEN)
    b2r = b2.reshape(NUM_EXPERTS, 1, DIM)

    kblocks = HIDDEN // HBLK
    out = pl.pallas_call(
        _ffff_kernel,
        grid=(NUM_EXPERTS, kblocks),
        in_specs=[
            pl.BlockSpec((n, dim), lambda e, k: (0, 0)),
            pl.BlockSpec((NUM_ROUTERS, dim), lambda e, k: (0, 0)),
            pl.BlockSpec((1, NUM_ROUTERS), lambda e, k: (0, 0)),
            pl.BlockSpec((1, dim, HBLK), lambda e, k: (e, 0, k)),
            pl.BlockSpec((1, 1, HBLK), lambda e, k: (e, 0, k)),
            pl.BlockSpec((1, HBLK, dim), lambda e, k: (e, k, 0)),
            pl.BlockSpec((1, 1, dim), lambda e, k: (e, 0, 0)),
        ],
        out_specs=pl.BlockSpec((n, dim), lambda e, k: (0, 0)),
        out_shape=jax.ShapeDtypeStruct((n, dim), jnp.float32),
        scratch_shapes=[pltpu.VMEM((n, NUM_EXPERTS), jnp.float32),
                        pltpu.VMEM((n, 1), jnp.float32)],
        compiler_params=pltpu.CompilerParams(
            dimension_semantics=("arbitrary", "arbitrary"),
            vmem_limit_bytes=100 * 1024 * 1024,
        ),
    )(flat_x, router_w, rb2d, w1, b1r, w2, b2r)
    return out.reshape(batch, seq, dim)


# speedup vs baseline: 1.0251x; 1.0251x over previous
import jax
import jax.numpy as jnp
from jax.experimental import pallas as pl
from jax.experimental.pallas import tpu as pltpu

DIM = 1024
DEPTH = 3
NUM_EXPERTS = 2 ** DEPTH
NUM_ROUTERS = NUM_EXPERTS - 1
HIDDEN = DIM * 4
HBLK = 1024


def _ffff_kernel(x_ref, rw_ref, rb_ref, w1_ref, b1_ref, w2_ref, b2_ref,
                 out_ref, p_ref, pe_ref):
    e = pl.program_id(0)
    k = pl.program_id(1)

    @pl.when((e == 0) & (k == 0))
    def _init():
        x = x_ref[...]
        logits = jax.lax.dot_general(
            x, rw_ref[...], (((1,), (1,)), ((), ())),
            preferred_element_type=jnp.float32) + rb_ref[...]
        s = jax.nn.sigmoid(logits)
        cols = []
        for ei in range(NUM_EXPERTS):
            bit0 = (ei >> 2) & 1
            bit1 = (ei >> 1) & 1
            bit2 = ei & 1
            c1 = 1 + (ei >> 2)
            c2 = 3 + (ei >> 1)
            t0 = s[:, 0:1] if bit0 else 1.0 - s[:, 0:1]
            t1 = s[:, c1:c1 + 1] if bit1 else 1.0 - s[:, c1:c1 + 1]
            t2 = s[:, c2:c2 + 1] if bit2 else 1.0 - s[:, c2:c2 + 1]
            cols.append(t0 * t1 * t2)
        p_ref[...] = jnp.concatenate(cols, axis=1)
        out_ref[...] = jnp.zeros_like(out_ref)

    @pl.when(k == 0)
    def _per_expert():
        lanes = jax.lax.broadcasted_iota(jnp.int32, (1, NUM_EXPERTS), 1)
        pe = jnp.sum(p_ref[...] * (lanes == e).astype(jnp.float32),
                     axis=1, keepdims=True)
        pe_ref[...] = pe
        out_ref[...] += pe * b2_ref[0]

    x = x_ref[...]
    pe = pe_ref[...]
    half = HBLK // 2
    partials = []
    for i in range(2):
        sl = slice(i * half, (i + 1) * half)
        h = jnp.dot(x, w1_ref[0, :, sl], preferred_element_type=jnp.float32)
        h = h + b1_ref[0, :, sl]
        h = 0.5 * h * (1.0 + jax.lax.erf(h * 0.7071067811865476))
        h = h * pe
        partials.append(jnp.dot(h, w2_ref[0, sl, :],
                                preferred_element_type=jnp.float32))
    out_ref[...] += partials[0] + partials[1]


def kernel(x, router_w, router_b, w1, b1, w2, b2):
    batch, seq, dim = x.shape
    n = batch * seq
    flat_x = x.reshape(n, dim)
    rb2d = router_b.reshape(1, NUM_ROUTERS)
    b1r = b1.reshape(NUM_EXPERTS, 1, HIDDEN)
    b2r = b2.reshape(NUM_EXPERTS, 1, DIM)

    kblocks = HIDDEN // HBLK
    out = pl.pallas_call(
        _ffff_kernel,
        grid=(NUM_EXPERTS, kblocks),
        in_specs=[
            pl.BlockSpec((n, dim), lambda e, k: (0, 0)),
            pl.BlockSpec((NUM_ROUTERS, dim), lambda e, k: (0, 0)),
            pl.BlockSpec((1, NUM_ROUTERS), lambda e, k: (0, 0)),
            pl.BlockSpec((1, dim, HBLK), lambda e, k: (e, 0, k)),
            pl.BlockSpec((1, 1, HBLK), lambda e, k: (e, 0, k)),
            pl.BlockSpec((1, HBLK, dim), lambda e, k: (e, k, 0)),
            pl.BlockSpec((1, 1, dim), lambda e, k: (e, 0, 0)),
        ],
        out_specs=pl.BlockSpec((n, dim), lambda e, k: (0, 0)),
        out_shape=jax.ShapeDtypeStruct((n, dim), jnp.float32),
        scratch_shapes=[pltpu.VMEM((n, NUM_EXPERTS), jnp.float32),
                        pltpu.VMEM((n, 1), jnp.float32)],
        compiler_params=pltpu.CompilerParams(
            dimension_semantics=("arbitrary", "arbitrary"),
            vmem_limit_bytes=100 * 1024 * 1024,
        ),
    )(flat_x, router_w, rb2d, w1, b1r, w2, b2r)
    return out.reshape(batch, seq, dim)
